# baseline (speedup 1.0000x reference)
import functools

import jax
import jax.numpy as jnp
from jax import lax
from jax.experimental import pallas as pl
from jax.experimental.pallas import tpu as pltpu
from jax.experimental.pallas import tpu_sc as plsc

_QB = 128
_NC = 2
_NW = 32
_BW = 256


def _argmin_body(pos_ref, pathT_ref, idx_ref):
    pos = pos_ref[...]
    pathT = pathT_ref[...]
    m = pathT.shape[1]
    nch = m // 128
    dot2 = lax.dot_general(pos + pos, pathT, (((1,), (0,)), ((), ())),
                           preferred_element_type=jnp.float32)
    pn = jnp.sum(pos * pos, axis=-1, keepdims=True)
    qn = jnp.sum(pathT * pathT, axis=0, keepdims=True)
    t = pn + qn

    best = jnp.full((_QB, 128), jnp.inf, jnp.float32)
    bchunk = jnp.zeros((_QB, 128), jnp.int32)
    for c in range(nch):
        d2 = lax.slice(t, (0, c * 128), (_QB, (c + 1) * 128)) - \
            lax.slice(dot2, (0, c * 128), (_QB, (c + 1) * 128))
        lt = d2 < best
        best = jnp.minimum(best, d2)
        bchunk = jnp.where(lt, c, bchunk)
    gmin = jnp.min(best, axis=1, keepdims=True)
    lane = lax.broadcasted_iota(jnp.int32, (_QB, 128), 1)
    full = bchunk * 128 + lane
    idx_ref[...] = jnp.min(jnp.where(best == gmin, full, jnp.int32(m)), axis=1)


def _nearest_idx(positions, path_points, interpret=False):
    q = positions.shape[0]
    m = path_points.shape[0]
    pathT = path_points.T
    return pl.pallas_call(
        _argmin_body,
        grid=(q // _QB,),
        in_specs=[
            pl.BlockSpec((_QB, 2), lambda i: (i, 0)),
            pl.BlockSpec((2, m), lambda i: (0, 0)),
        ],
        out_specs=pl.BlockSpec((_QB,), lambda i: (i,)),
        out_shape=jax.ShapeDtypeStruct((q,), jnp.int32),
        interpret=interpret,
    )(positions, pathT)


def _sc_gather(idx2d, tables, posx, posy):
    q = posx.shape[0]
    nt = len(tables)
    mesh = plsc.VectorSubcoreMesh(core_axis_name="c", subcore_axis_name="s")

    @functools.partial(
        pl.kernel,
        out_type=tuple(jax.ShapeDtypeStruct((q,), jnp.float32)
                       for _ in range(nt + 3)),
        mesh=mesh,
        scratch_types=(
            [pltpu.VMEM((2, 128), jnp.int32)]
            + [pltpu.VMEM((_BW,), jnp.float32) for _ in range(nt + 5)]
            + [pltpu.SemaphoreType.DMA]
        ),
    )
    def body(*refs):
        idx_hbm = refs[0]
        tab_hbms = refs[1:1 + nt]
        posx_hbm, posy_hbm = refs[1 + nt], refs[2 + nt]
        out_hbms = refs[3 + nt:3 + nt + nt]
        dx_hbm, dy_hbm, pr_hbm = refs[3 + 2 * nt:6 + 2 * nt]
        idx_v = refs[6 + 2 * nt]
        g_vs = refs[7 + 2 * nt:7 + 3 * nt]
        posx_v, posy_v, dx_v, dy_v, pr_v = refs[7 + 3 * nt:12 + 3 * nt]
        sem = refs[12 + 3 * nt]

        wid = lax.axis_index("s") * _NC + lax.axis_index("c")
        base = wid * _BW
        pltpu.sync_copy(idx_hbm.at[pl.ds(wid * 2, 2)], idx_v)
        pltpu.sync_copy(posx_hbm.at[pl.ds(base, _BW)], posx_v)
        pltpu.sync_copy(posy_hbm.at[pl.ds(base, _BW)], posy_v)
        cps = []
        for c in range(2):
            h = pl.ds(c * 128, 128)
            for t in range(nt):
                cps.append(pltpu.async_copy(tab_hbms[t].at[idx_v.at[c]],
                                            g_vs[t].at[h], sem))
        for cp in cps:
            cp.wait()

        pxg_v, pyg_v, nxg_v, nyg_v = g_vs[1], g_vs[2], g_vs[5], g_vs[6]

        def step(s, carry):
            sl = pl.ds(s * 16, 16)
            dx = posx_v[sl] - pxg_v[sl]
            dy = posy_v[sl] - pyg_v[sl]
            pr = dx * nxg_v[sl] + dy * nyg_v[sl]
            dx_v[sl] = dx
            dy_v[sl] = dy
            pr_v[sl] = pr
            return carry

        lax.fori_loop(0, _BW // 16, step, 0)
        dst = pl.ds(base, _BW)
        for t in range(nt):
            pltpu.sync_copy(g_vs[t], out_hbms[t].at[dst])
        pltpu.sync_copy(dx_v, dx_hbm.at[dst])
        pltpu.sync_copy(dy_v, dy_hbm.at[dst])
        pltpu.sync_copy(pr_v, pr_hbm.at[dst])

    return body(idx2d, *tables, posx, posy)


def kernel(positions, path_points, arclengths, tangents, normals, left_widths, right_widths):
    q = positions.shape[0]
    idx = _nearest_idx(positions, path_points)
    tables = (arclengths,
              path_points[:, 0], path_points[:, 1],
              tangents[:, 0], tangents[:, 1],
              normals[:, 0], normals[:, 1],
              left_widths, right_widths)
    (r, px, py, tx, ty, nx, ny, lw, rw, dx, dy, pr) = _sc_gather(
        idx.reshape(q // 128, 128), tables,
        positions[:, 0], positions[:, 1])
    closest_point_values = jnp.concatenate([px[:, None], py[:, None]], axis=1)
    closest_point_tangents = jnp.concatenate([tx[:, None], ty[:, None]], axis=1)
    closest_point_normals = jnp.concatenate([nx[:, None], ny[:, None]], axis=1)
    deltas = jnp.concatenate([dx[:, None], dy[:, None]], axis=1)
    return (r, closest_point_values, closest_point_tangents, closest_point_normals, deltas, pr, lw, rw)


# speedup vs baseline: 1.0707x; 1.0707x over previous
import functools

import jax
import jax.numpy as jnp
from jax import lax
from jax.experimental import pallas as pl
from jax.experimental.pallas import tpu as pltpu
from jax.experimental.pallas import tpu_sc as plsc

_QB = 256
_NC = 2
_NW = 32
_BW = 256


def _argmin_body(pos_ref, pathT_ref, idx_ref):
    pos = pos_ref[...]
    pathT = pathT_ref[...]
    m = pathT.shape[1]
    nch = m // 128
    dot2 = lax.dot_general(pos + pos, pathT, (((1,), (0,)), ((), ())),
                           preferred_element_type=jnp.float32)
    pn = jnp.sum(pos * pos, axis=-1, keepdims=True)
    qn = jnp.sum(pathT * pathT, axis=0, keepdims=True)
    t = pn + qn

    best = jnp.full((_QB, 128), jnp.inf, jnp.float32)
    bchunk = jnp.zeros((_QB, 128), jnp.int32)
    for c in range(nch):
        d2 = lax.slice(t, (0, c * 128), (_QB, (c + 1) * 128)) - \
            lax.slice(dot2, (0, c * 128), (_QB, (c + 1) * 128))
        lt = d2 < best
        best = jnp.minimum(best, d2)
        bchunk = jnp.where(lt, c, bchunk)
    gmin = jnp.min(best, axis=1, keepdims=True)
    lane = lax.broadcasted_iota(jnp.int32, (_QB, 128), 1)
    full = bchunk * 128 + lane
    idx_ref[...] = jnp.min(jnp.where(best == gmin, full, jnp.int32(m)), axis=1)


def _nearest_idx(positions, path_points, interpret=False):
    q = positions.shape[0]
    m = path_points.shape[0]
    pathT = path_points.T
    return pl.pallas_call(
        _argmin_body,
        grid=(q // _QB,),
        in_specs=[
            pl.BlockSpec((_QB, 2), lambda i: (i, 0)),
            pl.BlockSpec((2, m), lambda i: (0, 0)),
        ],
        out_specs=pl.BlockSpec((_QB,), lambda i: (i,)),
        out_shape=jax.ShapeDtypeStruct((q,), jnp.int32),
        interpret=interpret,
    )(positions, pathT)


def _sc_gather(idx2d, tables, posx, posy):
    q = posx.shape[0]
    nt = len(tables)
    mesh = plsc.VectorSubcoreMesh(core_axis_name="c", subcore_axis_name="s")

    @functools.partial(
        pl.kernel,
        out_type=tuple(jax.ShapeDtypeStruct((q,), jnp.float32)
                       for _ in range(nt + 3)),
        mesh=mesh,
        scratch_types=(
            [pltpu.VMEM((2, 128), jnp.int32)]
            + [pltpu.VMEM((_BW,), jnp.float32) for _ in range(nt + 5)]
            + [pltpu.SemaphoreType.DMA]
        ),
    )
    def body(*refs):
        idx_hbm = refs[0]
        tab_hbms = refs[1:1 + nt]
        posx_hbm, posy_hbm = refs[1 + nt], refs[2 + nt]
        out_hbms = refs[3 + nt:3 + nt + nt]
        dx_hbm, dy_hbm, pr_hbm = refs[3 + 2 * nt:6 + 2 * nt]
        idx_v = refs[6 + 2 * nt]
        g_vs = refs[7 + 2 * nt:7 + 3 * nt]
        posx_v, posy_v, dx_v, dy_v, pr_v = refs[7 + 3 * nt:12 + 3 * nt]
        sem = refs[12 + 3 * nt]

        wid = lax.axis_index("s") * _NC + lax.axis_index("c")
        base = wid * _BW
        pltpu.sync_copy(idx_hbm.at[pl.ds(wid * 2, 2)], idx_v)
        pltpu.sync_copy(posx_hbm.at[pl.ds(base, _BW)], posx_v)
        pltpu.sync_copy(posy_hbm.at[pl.ds(base, _BW)], posy_v)
        cps = []
        for c in range(2):
            h = pl.ds(c * 128, 128)
            for t in range(nt):
                cps.append(pltpu.async_copy(tab_hbms[t].at[idx_v.at[c]],
                                            g_vs[t].at[h], sem))
        for cp in cps:
            cp.wait()

        pxg_v, pyg_v, nxg_v, nyg_v = g_vs[1], g_vs[2], g_vs[5], g_vs[6]

        def step(s, carry):
            sl = pl.ds(s * 16, 16)
            dx = posx_v[sl] - pxg_v[sl]
            dy = posy_v[sl] - pyg_v[sl]
            pr = dx * nxg_v[sl] + dy * nyg_v[sl]
            dx_v[sl] = dx
            dy_v[sl] = dy
            pr_v[sl] = pr
            return carry

        lax.fori_loop(0, _BW // 16, step, 0)
        dst = pl.ds(base, _BW)
        for t in range(nt):
            pltpu.sync_copy(g_vs[t], out_hbms[t].at[dst])
        pltpu.sync_copy(dx_v, dx_hbm.at[dst])
        pltpu.sync_copy(dy_v, dy_hbm.at[dst])
        pltpu.sync_copy(pr_v, pr_hbm.at[dst])

    return body(idx2d, *tables, posx, posy)


def kernel(positions, path_points, arclengths, tangents, normals, left_widths, right_widths):
    q = positions.shape[0]
    idx = _nearest_idx(positions, path_points)
    tables = (arclengths,
              path_points[:, 0], path_points[:, 1],
              tangents[:, 0], tangents[:, 1],
              normals[:, 0], normals[:, 1],
              left_widths, right_widths)
    (r, px, py, tx, ty, nx, ny, lw, rw, dx, dy, pr) = _sc_gather(
        idx.reshape(q // 128, 128), tables,
        positions[:, 0], positions[:, 1])
    closest_point_values = jnp.concatenate([px[:, None], py[:, None]], axis=1)
    closest_point_tangents = jnp.concatenate([tx[:, None], ty[:, None]], axis=1)
    closest_point_normals = jnp.concatenate([nx[:, None], ny[:, None]], axis=1)
    deltas = jnp.concatenate([dx[:, None], dy[:, None]], axis=1)
    return (r, closest_point_values, closest_point_tangents, closest_point_normals, deltas, pr, lw, rw)


# speedup vs baseline: 1.1337x; 1.0588x over previous
import functools

import jax
import jax.numpy as jnp
from jax import lax
from jax.experimental import pallas as pl
from jax.experimental.pallas import tpu as pltpu
from jax.experimental.pallas import tpu_sc as plsc

_QB = 512
_NC = 2
_NW = 32
_BW = 256


def _argmin_body(pos_ref, pathT_ref, idx_ref):
    pos = pos_ref[...]
    pathT = pathT_ref[...]
    m = pathT.shape[1]
    nch = m // 128
    dot2 = lax.dot_general(pos + pos, pathT, (((1,), (0,)), ((), ())),
                           preferred_element_type=jnp.float32)
    pn = jnp.sum(pos * pos, axis=-1, keepdims=True)
    qn = jnp.sum(pathT * pathT, axis=0, keepdims=True)
    t = pn + qn

    best = jnp.full((_QB, 128), jnp.inf, jnp.float32)
    bchunk = jnp.zeros((_QB, 128), jnp.int32)
    for c in range(nch):
        d2 = lax.slice(t, (0, c * 128), (_QB, (c + 1) * 128)) - \
            lax.slice(dot2, (0, c * 128), (_QB, (c + 1) * 128))
        lt = d2 < best
        best = jnp.minimum(best, d2)
        bchunk = jnp.where(lt, c, bchunk)
    gmin = jnp.min(best, axis=1, keepdims=True)
    lane = lax.broadcasted_iota(jnp.int32, (_QB, 128), 1)
    full = bchunk * 128 + lane
    idx_ref[...] = jnp.min(jnp.where(best == gmin, full, jnp.int32(m)), axis=1)


def _nearest_idx(positions, path_points, interpret=False):
    q = positions.shape[0]
    m = path_points.shape[0]
    pathT = path_points.T
    return pl.pallas_call(
        _argmin_body,
        grid=(q // _QB,),
        in_specs=[
            pl.BlockSpec((_QB, 2), lambda i: (i, 0)),
            pl.BlockSpec((2, m), lambda i: (0, 0)),
        ],
        out_specs=pl.BlockSpec((_QB,), lambda i: (i,)),
        out_shape=jax.ShapeDtypeStruct((q,), jnp.int32),
        interpret=interpret,
    )(positions, pathT)


def _sc_gather(idx2d, tables, posx, posy):
    q = posx.shape[0]
    nt = len(tables)
    mesh = plsc.VectorSubcoreMesh(core_axis_name="c", subcore_axis_name="s")

    @functools.partial(
        pl.kernel,
        out_type=tuple(jax.ShapeDtypeStruct((q,), jnp.float32)
                       for _ in range(nt + 3)),
        mesh=mesh,
        scratch_types=(
            [pltpu.VMEM((2, 128), jnp.int32)]
            + [pltpu.VMEM((_BW,), jnp.float32) for _ in range(nt + 5)]
            + [pltpu.SemaphoreType.DMA]
        ),
    )
    def body(*refs):
        idx_hbm = refs[0]
        tab_hbms = refs[1:1 + nt]
        posx_hbm, posy_hbm = refs[1 + nt], refs[2 + nt]
        out_hbms = refs[3 + nt:3 + nt + nt]
        dx_hbm, dy_hbm, pr_hbm = refs[3 + 2 * nt:6 + 2 * nt]
        idx_v = refs[6 + 2 * nt]
        g_vs = refs[7 + 2 * nt:7 + 3 * nt]
        posx_v, posy_v, dx_v, dy_v, pr_v = refs[7 + 3 * nt:12 + 3 * nt]
        sem = refs[12 + 3 * nt]

        wid = lax.axis_index("s") * _NC + lax.axis_index("c")
        base = wid * _BW
        pltpu.sync_copy(idx_hbm.at[pl.ds(wid * 2, 2)], idx_v)
        pltpu.sync_copy(posx_hbm.at[pl.ds(base, _BW)], posx_v)
        pltpu.sync_copy(posy_hbm.at[pl.ds(base, _BW)], posy_v)
        cps = []
        for c in range(2):
            h = pl.ds(c * 128, 128)
            for t in range(nt):
                cps.append(pltpu.async_copy(tab_hbms[t].at[idx_v.at[c]],
                                            g_vs[t].at[h], sem))
        for cp in cps:
            cp.wait()

        pxg_v, pyg_v, nxg_v, nyg_v = g_vs[1], g_vs[2], g_vs[5], g_vs[6]

        def step(s, carry):
            sl = pl.ds(s * 16, 16)
            dx = posx_v[sl] - pxg_v[sl]
            dy = posy_v[sl] - pyg_v[sl]
            pr = dx * nxg_v[sl] + dy * nyg_v[sl]
            dx_v[sl] = dx
            dy_v[sl] = dy
            pr_v[sl] = pr
            return carry

        lax.fori_loop(0, _BW // 16, step, 0)
        dst = pl.ds(base, _BW)
        for t in range(nt):
            pltpu.sync_copy(g_vs[t], out_hbms[t].at[dst])
        pltpu.sync_copy(dx_v, dx_hbm.at[dst])
        pltpu.sync_copy(dy_v, dy_hbm.at[dst])
        pltpu.sync_copy(pr_v, pr_hbm.at[dst])

    return body(idx2d, *tables, posx, posy)


def kernel(positions, path_points, arclengths, tangents, normals, left_widths, right_widths):
    q = positions.shape[0]
    idx = _nearest_idx(positions, path_points)
    tables = (arclengths,
              path_points[:, 0], path_points[:, 1],
              tangents[:, 0], tangents[:, 1],
              normals[:, 0], normals[:, 1],
              left_widths, right_widths)
    (r, px, py, tx, ty, nx, ny, lw, rw, dx, dy, pr) = _sc_gather(
        idx.reshape(q // 128, 128), tables,
        positions[:, 0], positions[:, 1])
    closest_point_values = jnp.concatenate([px[:, None], py[:, None]], axis=1)
    closest_point_tangents = jnp.concatenate([tx[:, None], ty[:, None]], axis=1)
    closest_point_normals = jnp.concatenate([nx[:, None], ny[:, None]], axis=1)
    deltas = jnp.concatenate([dx[:, None], dy[:, None]], axis=1)
    return (r, closest_point_values, closest_point_tangents, closest_point_normals, deltas, pr, lw, rw)


# speedup vs baseline: 1.2076x; 1.0652x over previous
import functools

import jax
import jax.numpy as jnp
from jax import lax
from jax.experimental import pallas as pl
from jax.experimental.pallas import tpu as pltpu
from jax.experimental.pallas import tpu_sc as plsc

_QB = 1024
_NC = 2
_NW = 32
_BW = 256


def _argmin_body(pos_ref, pathT_ref, idx_ref):
    pos = pos_ref[...]
    pathT = pathT_ref[...]
    m = pathT.shape[1]
    nch = m // 128
    dot2 = lax.dot_general(pos + pos, pathT, (((1,), (0,)), ((), ())),
                           preferred_element_type=jnp.float32)
    pn = jnp.sum(pos * pos, axis=-1, keepdims=True)
    qn = jnp.sum(pathT * pathT, axis=0, keepdims=True)
    t = pn + qn

    best = jnp.full((_QB, 128), jnp.inf, jnp.float32)
    bchunk = jnp.zeros((_QB, 128), jnp.int32)
    for c in range(nch):
        d2 = lax.slice(t, (0, c * 128), (_QB, (c + 1) * 128)) - \
            lax.slice(dot2, (0, c * 128), (_QB, (c + 1) * 128))
        lt = d2 < best
        best = jnp.minimum(best, d2)
        bchunk = jnp.where(lt, c, bchunk)
    gmin = jnp.min(best, axis=1, keepdims=True)
    lane = lax.broadcasted_iota(jnp.int32, (_QB, 128), 1)
    full = bchunk * 128 + lane
    idx_ref[...] = jnp.min(jnp.where(best == gmin, full, jnp.int32(m)), axis=1)


def _nearest_idx(positions, path_points, interpret=False):
    q = positions.shape[0]
    m = path_points.shape[0]
    pathT = path_points.T
    return pl.pallas_call(
        _argmin_body,
        grid=(q // _QB,),
        in_specs=[
            pl.BlockSpec((_QB, 2), lambda i: (i, 0)),
            pl.BlockSpec((2, m), lambda i: (0, 0)),
        ],
        out_specs=pl.BlockSpec((_QB,), lambda i: (i,)),
        out_shape=jax.ShapeDtypeStruct((q,), jnp.int32),
        interpret=interpret,
    )(positions, pathT)


def _sc_gather(idx2d, tables, posx, posy):
    q = posx.shape[0]
    nt = len(tables)
    mesh = plsc.VectorSubcoreMesh(core_axis_name="c", subcore_axis_name="s")

    @functools.partial(
        pl.kernel,
        out_type=tuple(jax.ShapeDtypeStruct((q,), jnp.float32)
                       for _ in range(nt + 3)),
        mesh=mesh,
        scratch_types=(
            [pltpu.VMEM((2, 128), jnp.int32)]
            + [pltpu.VMEM((_BW,), jnp.float32) for _ in range(nt + 5)]
            + [pltpu.SemaphoreType.DMA]
        ),
    )
    def body(*refs):
        idx_hbm = refs[0]
        tab_hbms = refs[1:1 + nt]
        posx_hbm, posy_hbm = refs[1 + nt], refs[2 + nt]
        out_hbms = refs[3 + nt:3 + nt + nt]
        dx_hbm, dy_hbm, pr_hbm = refs[3 + 2 * nt:6 + 2 * nt]
        idx_v = refs[6 + 2 * nt]
        g_vs = refs[7 + 2 * nt:7 + 3 * nt]
        posx_v, posy_v, dx_v, dy_v, pr_v = refs[7 + 3 * nt:12 + 3 * nt]
        sem = refs[12 + 3 * nt]

        wid = lax.axis_index("s") * _NC + lax.axis_index("c")
        base = wid * _BW
        pltpu.sync_copy(idx_hbm.at[pl.ds(wid * 2, 2)], idx_v)
        pltpu.sync_copy(posx_hbm.at[pl.ds(base, _BW)], posx_v)
        pltpu.sync_copy(posy_hbm.at[pl.ds(base, _BW)], posy_v)
        cps = []
        for c in range(2):
            h = pl.ds(c * 128, 128)
            for t in range(nt):
                cps.append(pltpu.async_copy(tab_hbms[t].at[idx_v.at[c]],
                                            g_vs[t].at[h], sem))
        for cp in cps:
            cp.wait()

        pxg_v, pyg_v, nxg_v, nyg_v = g_vs[1], g_vs[2], g_vs[5], g_vs[6]

        def step(s, carry):
            sl = pl.ds(s * 16, 16)
            dx = posx_v[sl] - pxg_v[sl]
            dy = posy_v[sl] - pyg_v[sl]
            pr = dx * nxg_v[sl] + dy * nyg_v[sl]
            dx_v[sl] = dx
            dy_v[sl] = dy
            pr_v[sl] = pr
            return carry

        lax.fori_loop(0, _BW // 16, step, 0)
        dst = pl.ds(base, _BW)
        for t in range(nt):
            pltpu.sync_copy(g_vs[t], out_hbms[t].at[dst])
        pltpu.sync_copy(dx_v, dx_hbm.at[dst])
        pltpu.sync_copy(dy_v, dy_hbm.at[dst])
        pltpu.sync_copy(pr_v, pr_hbm.at[dst])

    return body(idx2d, *tables, posx, posy)


def kernel(positions, path_points, arclengths, tangents, normals, left_widths, right_widths):
    q = positions.shape[0]
    idx = _nearest_idx(positions, path_points)
    tables = (arclengths,
              path_points[:, 0], path_points[:, 1],
              tangents[:, 0], tangents[:, 1],
              normals[:, 0], normals[:, 1],
              left_widths, right_widths)
    (r, px, py, tx, ty, nx, ny, lw, rw, dx, dy, pr) = _sc_gather(
        idx.reshape(q // 128, 128), tables,
        positions[:, 0], positions[:, 1])
    closest_point_values = jnp.concatenate([px[:, None], py[:, None]], axis=1)
    closest_point_tangents = jnp.concatenate([tx[:, None], ty[:, None]], axis=1)
    closest_point_normals = jnp.concatenate([nx[:, None], ny[:, None]], axis=1)
    deltas = jnp.concatenate([dx[:, None], dy[:, None]], axis=1)
    return (r, closest_point_values, closest_point_tangents, closest_point_normals, deltas, pr, lw, rw)


# speedup vs baseline: 1.2166x; 1.0074x over previous
import functools

import jax
import jax.numpy as jnp
from jax import lax
from jax.experimental import pallas as pl
from jax.experimental.pallas import tpu as pltpu
from jax.experimental.pallas import tpu_sc as plsc

_QB = 2048
_NC = 2
_NW = 32
_BW = 256


def _argmin_body(pos_ref, pathT_ref, idx_ref):
    pos = pos_ref[...]
    pathT = pathT_ref[...]
    m = pathT.shape[1]
    nch = m // 128
    dot2 = lax.dot_general(pos + pos, pathT, (((1,), (0,)), ((), ())),
                           preferred_element_type=jnp.float32)
    pn = jnp.sum(pos * pos, axis=-1, keepdims=True)
    qn = jnp.sum(pathT * pathT, axis=0, keepdims=True)
    t = pn + qn

    best = jnp.full((_QB, 128), jnp.inf, jnp.float32)
    bchunk = jnp.zeros((_QB, 128), jnp.int32)
    for c in range(nch):
        d2 = lax.slice(t, (0, c * 128), (_QB, (c + 1) * 128)) - \
            lax.slice(dot2, (0, c * 128), (_QB, (c + 1) * 128))
        lt = d2 < best
        best = jnp.minimum(best, d2)
        bchunk = jnp.where(lt, c, bchunk)
    gmin = jnp.min(best, axis=1, keepdims=True)
    lane = lax.broadcasted_iota(jnp.int32, (_QB, 128), 1)
    full = bchunk * 128 + lane
    idx_ref[...] = jnp.min(jnp.where(best == gmin, full, jnp.int32(m)), axis=1)


def _nearest_idx(positions, path_points, interpret=False):
    q = positions.shape[0]
    m = path_points.shape[0]
    pathT = path_points.T
    return pl.pallas_call(
        _argmin_body,
        grid=(q // _QB,),
        in_specs=[
            pl.BlockSpec((_QB, 2), lambda i: (i, 0)),
            pl.BlockSpec((2, m), lambda i: (0, 0)),
        ],
        out_specs=pl.BlockSpec((_QB,), lambda i: (i,)),
        out_shape=jax.ShapeDtypeStruct((q,), jnp.int32),
        interpret=interpret,
    )(positions, pathT)


def _sc_gather(idx2d, tables, posx, posy):
    q = posx.shape[0]
    nt = len(tables)
    mesh = plsc.VectorSubcoreMesh(core_axis_name="c", subcore_axis_name="s")

    @functools.partial(
        pl.kernel,
        out_type=tuple(jax.ShapeDtypeStruct((q,), jnp.float32)
                       for _ in range(nt + 3)),
        mesh=mesh,
        scratch_types=(
            [pltpu.VMEM((2, 128), jnp.int32)]
            + [pltpu.VMEM((_BW,), jnp.float32) for _ in range(nt + 5)]
            + [pltpu.SemaphoreType.DMA]
        ),
    )
    def body(*refs):
        idx_hbm = refs[0]
        tab_hbms = refs[1:1 + nt]
        posx_hbm, posy_hbm = refs[1 + nt], refs[2 + nt]
        out_hbms = refs[3 + nt:3 + nt + nt]
        dx_hbm, dy_hbm, pr_hbm = refs[3 + 2 * nt:6 + 2 * nt]
        idx_v = refs[6 + 2 * nt]
        g_vs = refs[7 + 2 * nt:7 + 3 * nt]
        posx_v, posy_v, dx_v, dy_v, pr_v = refs[7 + 3 * nt:12 + 3 * nt]
        sem = refs[12 + 3 * nt]

        wid = lax.axis_index("s") * _NC + lax.axis_index("c")
        base = wid * _BW
        pltpu.sync_copy(idx_hbm.at[pl.ds(wid * 2, 2)], idx_v)
        pltpu.sync_copy(posx_hbm.at[pl.ds(base, _BW)], posx_v)
        pltpu.sync_copy(posy_hbm.at[pl.ds(base, _BW)], posy_v)
        cps = []
        for c in range(2):
            h = pl.ds(c * 128, 128)
            for t in range(nt):
                cps.append(pltpu.async_copy(tab_hbms[t].at[idx_v.at[c]],
                                            g_vs[t].at[h], sem))
        for cp in cps:
            cp.wait()

        pxg_v, pyg_v, nxg_v, nyg_v = g_vs[1], g_vs[2], g_vs[5], g_vs[6]

        def step(s, carry):
            sl = pl.ds(s * 16, 16)
            dx = posx_v[sl] - pxg_v[sl]
            dy = posy_v[sl] - pyg_v[sl]
            pr = dx * nxg_v[sl] + dy * nyg_v[sl]
            dx_v[sl] = dx
            dy_v[sl] = dy
            pr_v[sl] = pr
            return carry

        lax.fori_loop(0, _BW // 16, step, 0)
        dst = pl.ds(base, _BW)
        for t in range(nt):
            pltpu.sync_copy(g_vs[t], out_hbms[t].at[dst])
        pltpu.sync_copy(dx_v, dx_hbm.at[dst])
        pltpu.sync_copy(dy_v, dy_hbm.at[dst])
        pltpu.sync_copy(pr_v, pr_hbm.at[dst])

    return body(idx2d, *tables, posx, posy)


def kernel(positions, path_points, arclengths, tangents, normals, left_widths, right_widths):
    q = positions.shape[0]
    idx = _nearest_idx(positions, path_points)
    tables = (arclengths,
              path_points[:, 0], path_points[:, 1],
              tangents[:, 0], tangents[:, 1],
              normals[:, 0], normals[:, 1],
              left_widths, right_widths)
    (r, px, py, tx, ty, nx, ny, lw, rw, dx, dy, pr) = _sc_gather(
        idx.reshape(q // 128, 128), tables,
        positions[:, 0], positions[:, 1])
    closest_point_values = jnp.concatenate([px[:, None], py[:, None]], axis=1)
    closest_point_tangents = jnp.concatenate([tx[:, None], ty[:, None]], axis=1)
    closest_point_normals = jnp.concatenate([nx[:, None], ny[:, None]], axis=1)
    deltas = jnp.concatenate([dx[:, None], dy[:, None]], axis=1)
    return (r, closest_point_values, closest_point_tangents, closest_point_normals, deltas, pr, lw, rw)
